# baseline (speedup 1.0000x reference)
import functools

import jax
import jax.numpy as jnp
from jax import lax
from jax.experimental import pallas as pl
from jax.experimental.pallas import tpu as pltpu
from jax.experimental.pallas import tpu_sc as plsc

_N = 10000
_E = 320000
_D = 128
_H = 128
_A = 16
_B = 64

_RB = 512
_NP = 10240
_NPB = _NP // _RB

_NC, _NS = 2, 16
_NW = _NC * _NS
_EB = 128
_NBW = 80
_NBW0 = 80
_NBW1 = 80
_IDXC = 40
_QB = _EB // 4
_EP = _NW * _NBW * _EB
_ZR = _NP // _NS

_mesh = plsc.VectorSubcoreMesh(
    core_axis_name="c", subcore_axis_name="s",
    num_cores=_NC, num_subcores=_NS)


def _sc_deg(dst2d, ones_eb, zeros_zr):
  @functools.partial(
      pl.kernel,
      out_type=jax.ShapeDtypeStruct((_NC * _NP,), jnp.float32),
      mesh=_mesh,
      scratch_types=[
          pltpu.VMEM((_NBW, _EB), jnp.int32),
          pltpu.VMEM((_EB,), jnp.float32),
          pltpu.VMEM_SHARED((_NP,), jnp.float32),
      ],
  )
  def run(dst_hbm, ones_hbm, z_hbm, out_hbm, dstv, onesv, acc):
    c = lax.axis_index("c")
    s = lax.axis_index("s")
    w = c * _NS + s
    pltpu.sync_copy(z_hbm, acc.at[pl.ds(s * _ZR, _ZR)])
    pltpu.sync_copy(dst_hbm.at[pl.ds(w * _NBW, _NBW)], dstv)
    pltpu.sync_copy(ones_hbm, onesv)
    plsc.subcore_barrier()

    def body(j, carry):
      pltpu.sync_copy(onesv, acc.at[dstv.at[j]], add=True)
      return carry

    lax.fori_loop(0, _NBW, body, 0)
    plsc.subcore_barrier()
    pltpu.sync_copy(acc.at[pl.ds(s * _ZR, _ZR)],
                    out_hbm.at[pl.ds(c * _NP + s * _ZR, _ZR)])

  return run(dst2d, ones_eb, zeros_zr)


def _sc_agg(ht, src2d, dst2d, zrows):
  @functools.partial(
      pl.kernel,
      out_type=jax.ShapeDtypeStruct((_NC, _NP, _H), jnp.float32),
      mesh=_mesh,
      scratch_types=[
          pltpu.VMEM((_IDXC, _EB), jnp.int32),
          pltpu.VMEM((_IDXC, _EB), jnp.int32),
          pltpu.VMEM((2 * _EB, _H), jnp.float32),
          pltpu.VMEM_SHARED((_NP, _H), jnp.float32),
          [pltpu.SemaphoreType.DMA] * 8,
      ],
  )
  def run(ht_hbm, src_hbm, dst_hbm, z_hbm, out_hbm, srcv, dstv, rb,
          acc, sems):
    c = lax.axis_index("c")
    s = lax.axis_index("s")
    base = jnp.where(c == 0, s * _NBW0, _NS * _NBW0 + s * _NBW1)
    nchunks = jnp.where(c == 0, _NBW0 // _IDXC, _NBW1 // _IDXC)
    pltpu.sync_copy(z_hbm, acc.at[pl.ds(s * _ZR, _ZR)])
    plsc.subcore_barrier()

    def issue(jj, par):
      s0 = 4 * par
      for q in range(4):
        pltpu.async_copy(ht_hbm.at[srcv.at[jj, pl.ds(q * _QB, _QB)]],
                         rb.at[pl.ds((s0 + q) * _QB, _QB)], sems[s0 + q])

    def drain_scatter(jj, par):
      s0 = 4 * par
      for q in range(4):
        pltpu.make_async_copy(ht_hbm.at[srcv.at[jj, pl.ds(q * _QB, _QB)]],
                              rb.at[pl.ds((s0 + q) * _QB, _QB)],
                              sems[s0 + q]).wait()
      pltpu.sync_copy(rb.at[pl.ds(s0 * _QB, _EB)], acc.at[dstv.at[jj]],
                      add=True)

    def chunk(p, carry):
      off = base + p * _IDXC
      pltpu.sync_copy(src_hbm.at[pl.ds(off, _IDXC)], srcv)
      pltpu.sync_copy(dst_hbm.at[pl.ds(off, _IDXC)], dstv)
      issue(0, 0)
      issue(1, 1)

      def group(g, carry2):
        b0 = 4 * g
        for k in range(4):
          jj = b0 + k
          drain_scatter(jj, k % 2)

          @pl.when(jj + 2 < _IDXC)
          def _():
            issue(jj + 2, k % 2)
        return carry2

      lax.fori_loop(0, _IDXC // 4, group, 0)
      return carry

    lax.fori_loop(0, nchunks, chunk, 0)
    plsc.subcore_barrier()
    pltpu.sync_copy(acc.at[pl.ds(s * _ZR, _ZR)],
                    out_hbm.at[c].at[pl.ds(s * _ZR, _ZR)])

  return run(ht, src2d, dst2d, zrows)


def _dis(d0, d1, i):
  dis = lax.rsqrt(1.0 + d0 + d1)
  row = lax.broadcasted_iota(jnp.int32, (_RB, 1), 0) + i * _RB
  return jnp.where(row < _N, dis, 0.0)


def _tc1_body(x_ref, w_ref, d0_ref, d1_ref, g_ref, ht_ref):
  i = pl.program_id(0)
  dis = _dis(d0_ref[...], d1_ref[...], i)
  g = jnp.dot(x_ref[...], w_ref[...], preferred_element_type=jnp.float32)
  g_ref[...] = g
  ht_ref[...] = g * dis


def _tc2_body(a0_ref, a1_ref, g1_ref, d0_ref, d1_ref, w_ref, b_ref,
              g2_ref, ht_ref):
  i = pl.program_id(0)
  dis = _dis(d0_ref[...], d1_ref[...], i)
  h1 = jnp.maximum(
      dis * (a0_ref[...] + a1_ref[...]) + dis * dis * g1_ref[...] + b_ref[...],
      0.0)
  g2 = jnp.dot(h1, w_ref[...], preferred_element_type=jnp.float32)
  g2_ref[...] = g2
  ht_ref[...] = g2 * dis


def _tc3_body(q0_ref, q1_ref, g2_ref, d0_ref, d1_ref, b_ref, batch_ref,
              af_ref, fw1a_ref, fw1b_ref, fb1_ref, fw2_ref, fb2_ref,
              out_ref, sums, cnts):
  i = pl.program_id(0)
  dis = _dis(d0_ref[...], d1_ref[...], i)
  h2 = jnp.maximum(
      dis * (q0_ref[...] + q1_ref[...]) + dis * dis * g2_ref[...] + b_ref[...],
      0.0)
  row = lax.broadcasted_iota(jnp.int32, (_RB, 1), 0) + i * _RB
  h2 = jnp.where(row < _N, h2, 0.0)
  oh = (batch_ref[...] ==
        lax.broadcasted_iota(jnp.int32, (_B, _RB), 0)).astype(jnp.float32)
  psum = jnp.dot(oh, h2, preferred_element_type=jnp.float32)
  pcnt = jnp.sum(oh, axis=1, keepdims=True)

  @pl.when(i == 0)
  def _():
    sums[...] = jnp.zeros_like(sums)
    cnts[...] = jnp.zeros_like(cnts)

  sums[...] = sums[...] + psum
  cnts[...] = cnts[...] + pcnt

  @pl.when(i == _NPB - 1)
  def _():
    pooled = sums[...] / jnp.maximum(cnts[...], 1.0)
    z = jnp.maximum(
        jnp.dot(pooled, fw1a_ref[...], preferred_element_type=jnp.float32)
        + jnp.dot(af_ref[...], fw1b_ref[...], preferred_element_type=jnp.float32)
        + fb1_ref[...], 0.0)
    out_ref[...] = (jnp.dot(z, fw2_ref[...], preferred_element_type=jnp.float32)
                    + fb2_ref[...])


def _row_spec():
  return pl.BlockSpec((_RB, _H), lambda i: (i, 0))


def _col_spec():
  return pl.BlockSpec((_RB, 1), lambda i: (i, 0))


def _full_spec(shape):
  return pl.BlockSpec(shape, lambda i: tuple(0 for _ in shape))


def kernel(x, edge_index, batch, additional_features,
           W1, b1, W2, b2, FW1, Fb1, FW2, Fb2):
  f32 = jnp.float32
  xp = jnp.pad(x, ((0, _NP - _N), (0, 0)))
  pad_i = jnp.arange(_EP - _E, dtype=jnp.int32)
  src_pad = _N + pad_i % (_NP - _N)
  dst_pad = _N + (pad_i + 120) % (_NP - _N)
  src2d = jnp.concatenate([edge_index[0], src_pad]).reshape(_NW * _NBW, _EB)
  dst2d = jnp.concatenate([edge_index[1], dst_pad]).reshape(_NW * _NBW, _EB)
  batch2 = jnp.pad(batch, (0, _NP - _N), constant_values=_B)[None, :]
  ones_eb = jnp.ones((_EB,), f32)
  zeros_zr = jnp.zeros((_ZR,), f32)
  zrows = jnp.zeros((_ZR, _H), f32)
  b1r = b1[None, :]
  b2r = b2[None, :]
  fw1a = FW1[:_H]
  fw1b = FW1[_H:]
  fb1r = Fb1[None, :]
  fw2p = jnp.pad(FW2, ((0, 0), (0, _H - FW2.shape[1])))
  fb2p = jnp.pad(Fb2, (0, _H - Fb2.shape[0]))[None, :]

  degp = _sc_deg(dst2d, ones_eb, zeros_zr).reshape(_NC, _NP)
  d0c = degp[0][:, None]
  d1c = degp[1][:, None]

  g1, ht1 = pl.pallas_call(
      _tc1_body,
      grid=(_NPB,),
      in_specs=[_row_spec(), _full_spec((_D, _H)), _col_spec(), _col_spec()],
      out_specs=[_row_spec(), _row_spec()],
      out_shape=[jax.ShapeDtypeStruct((_NP, _H), f32)] * 2,
  )(xp, W1, d0c, d1c)

  aggp = _sc_agg(ht1, src2d, dst2d, zrows)

  g2, ht2 = pl.pallas_call(
      _tc2_body,
      grid=(_NPB,),
      in_specs=[_row_spec(), _row_spec(), _row_spec(), _col_spec(),
                _col_spec(), _full_spec((_H, _H)), _full_spec((1, _H))],
      out_specs=[_row_spec(), _row_spec()],
      out_shape=[jax.ShapeDtypeStruct((_NP, _H), f32)] * 2,
  )(aggp[0], aggp[1], g1, d0c, d1c, W2, b1r)

  qp = _sc_agg(ht2, src2d, dst2d, zrows)

  out_full = pl.pallas_call(
      _tc3_body,
      grid=(_NPB,),
      in_specs=[_row_spec(), _row_spec(), _row_spec(), _col_spec(),
                _col_spec(), _full_spec((1, _H)),
                pl.BlockSpec((1, _RB), lambda i: (0, i)),
                _full_spec((_B, _A)), _full_spec((_H, _H)),
                _full_spec((_A, _H)), _full_spec((1, _H)),
                _full_spec((_H, _H)), _full_spec((1, _H))],
      out_specs=pl.BlockSpec((_B, _H), lambda i: (0, 0)),
      out_shape=jax.ShapeDtypeStruct((_B, _H), f32),
      scratch_shapes=[pltpu.VMEM((_B, _H), f32), pltpu.VMEM((_B, 1), f32)],
  )(qp[0], qp[1], g2, d0c, d1c, b2r, batch2, additional_features,
    fw1a, fw1b, fb1r, fw2p, fb2p)

  return out_full[:, :FW2.shape[1]]


# speedup vs baseline: 1.0620x; 1.0620x over previous
import functools

import jax
import jax.numpy as jnp
from jax import lax
from jax.experimental import pallas as pl
from jax.experimental.pallas import tpu as pltpu
from jax.experimental.pallas import tpu_sc as plsc

_N = 10000
_E = 320000
_D = 128
_H = 128
_A = 16
_B = 64

_RB = 512
_NP = 10240
_NPB = _NP // _RB

_NC, _NS = 2, 16
_NW = _NC * _NS
_EB = 128
_NBW = 80
_NBW0 = 80
_NBW1 = 80
_IDXC = 40
_QB = _EB // 2
_EP = _NW * _NBW * _EB
_ZR = _NP // _NS

_mesh = plsc.VectorSubcoreMesh(
    core_axis_name="c", subcore_axis_name="s",
    num_cores=_NC, num_subcores=_NS)


def _sc_deg(dst2d, ones_eb, zeros_zr):
  @functools.partial(
      pl.kernel,
      out_type=[jax.ShapeDtypeStruct((_NP, _H), jnp.float32)] * 2,
      mesh=_mesh,
      scratch_types=[
          pltpu.VMEM((_NBW, _EB), jnp.int32),
          pltpu.VMEM((_EB,), jnp.float32),
          pltpu.VMEM((_ZR,), jnp.float32),
          pltpu.VMEM((_ZR, _H), jnp.float32),
          pltpu.VMEM_SHARED((_NP,), jnp.float32),
      ],
  )
  def run(dst_hbm, ones_hbm, z_hbm, out0_hbm, out1_hbm,
          dstv, onesv, dv, bc, acc):
    c = lax.axis_index("c")
    s = lax.axis_index("s")
    w = c * _NS + s
    pltpu.sync_copy(z_hbm, acc.at[pl.ds(s * _ZR, _ZR)])
    pltpu.sync_copy(dst_hbm.at[pl.ds(w * _NBW, _NBW)], dstv)
    pltpu.sync_copy(ones_hbm, onesv)
    plsc.subcore_barrier()

    def body(j, carry):
      pltpu.sync_copy(onesv, acc.at[dstv.at[j]], add=True)
      return carry

    lax.fori_loop(0, _NBW, body, 0)
    plsc.subcore_barrier()
    pltpu.sync_copy(acc.at[pl.ds(s * _ZR, _ZR)], dv)

    def brow(rg, carry):
      vals = dv[pl.ds(16 * rg, 16)]
      for k in range(16):
        row = jnp.full((16,), vals[k], jnp.float32)
        for q in range(8):
          bc[16 * rg + k, pl.ds(16 * q, 16)] = row
      return carry

    lax.fori_loop(0, _ZR // 16, brow, 0)

    @pl.when(c == 0)
    def _():
      pltpu.sync_copy(bc, out0_hbm.at[pl.ds(s * _ZR, _ZR)])

    @pl.when(c == 1)
    def _():
      pltpu.sync_copy(bc, out1_hbm.at[pl.ds(s * _ZR, _ZR)])

  return run(dst2d, ones_eb, zeros_zr)


def _sc_agg(ht, src2d, dst2d, zrows):
  @functools.partial(
      pl.kernel,
      out_type=[jax.ShapeDtypeStruct((_NP, _H), jnp.float32)] * 2,
      mesh=_mesh,
      scratch_types=[
          pltpu.VMEM((_IDXC, _EB), jnp.int32),
          pltpu.VMEM((_IDXC, _EB), jnp.int32),
          pltpu.VMEM((2 * _EB, _H), jnp.float32),
          pltpu.VMEM_SHARED((_NP, _H), jnp.float32),
          [pltpu.SemaphoreType.DMA] * 4,
      ],
  )
  def run(ht_hbm, src_hbm, dst_hbm, z_hbm, out0_hbm, out1_hbm,
          srcv, dstv, rb, acc, sems):
    c = lax.axis_index("c")
    s = lax.axis_index("s")
    base = jnp.where(c == 0, s * _NBW0, _NS * _NBW0 + s * _NBW1)
    nchunks = jnp.where(c == 0, _NBW0 // _IDXC, _NBW1 // _IDXC)
    pltpu.sync_copy(z_hbm, acc.at[pl.ds(s * _ZR, _ZR)])
    plsc.subcore_barrier()

    def issue(jj, par):
      s0 = 2 * par
      for q in range(2):
        pltpu.async_copy(ht_hbm.at[srcv.at[jj, pl.ds(q * _QB, _QB)]],
                         rb.at[pl.ds((s0 + q) * _QB, _QB)], sems[s0 + q])

    def drain_scatter(jj, par):
      s0 = 2 * par
      for q in range(2):
        pltpu.make_async_copy(ht_hbm.at[srcv.at[jj, pl.ds(q * _QB, _QB)]],
                              rb.at[pl.ds((s0 + q) * _QB, _QB)],
                              sems[s0 + q]).wait()
      pltpu.sync_copy(rb.at[pl.ds(s0 * _QB, _EB)], acc.at[dstv.at[jj]],
                      add=True)

    def chunk(p, carry):
      off = base + p * _IDXC
      pltpu.sync_copy(src_hbm.at[pl.ds(off, _IDXC)], srcv)
      pltpu.sync_copy(dst_hbm.at[pl.ds(off, _IDXC)], dstv)
      issue(0, 0)
      issue(1, 1)

      def group(g, carry2):
        b0 = 4 * g
        for k in range(4):
          jj = b0 + k
          drain_scatter(jj, k % 2)

          @pl.when(jj + 2 < _IDXC)
          def _():
            issue(jj + 2, k % 2)
        return carry2

      lax.fori_loop(0, _IDXC // 4, group, 0)
      return carry

    lax.fori_loop(0, nchunks, chunk, 0)
    plsc.subcore_barrier()

    @pl.when(c == 0)
    def _():
      pltpu.sync_copy(acc.at[pl.ds(s * _ZR, _ZR)],
                      out0_hbm.at[pl.ds(s * _ZR, _ZR)])

    @pl.when(c == 1)
    def _():
      pltpu.sync_copy(acc.at[pl.ds(s * _ZR, _ZR)],
                      out1_hbm.at[pl.ds(s * _ZR, _ZR)])

  return run(ht, src2d, dst2d, zrows)


def _dis(p0, p1, i):
  dis = lax.rsqrt(1.0 + p0 + p1)
  row = lax.broadcasted_iota(jnp.int32, (_RB, 1), 0) + i * _RB
  return jnp.where(row < _N, dis, 0.0)


def _tc1_body(x_ref, w_ref, d0_ref, d1_ref, g_ref, ht_ref):
  i = pl.program_id(0)
  dis = _dis(d0_ref[...], d1_ref[...], i)
  g = jnp.dot(x_ref[...], w_ref[...], preferred_element_type=jnp.float32)
  g_ref[...] = g
  ht_ref[...] = g * dis


def _tc2_body(a0_ref, a1_ref, g1_ref, d0_ref, d1_ref, w_ref, b_ref,
              g2_ref, ht_ref):
  i = pl.program_id(0)
  dis = _dis(d0_ref[...], d1_ref[...], i)
  h1 = jnp.maximum(
      dis * (a0_ref[...] + a1_ref[...]) + dis * dis * g1_ref[...] + b_ref[...],
      0.0)
  g2 = jnp.dot(h1, w_ref[...], preferred_element_type=jnp.float32)
  g2_ref[...] = g2
  ht_ref[...] = g2 * dis


def _tc3_body(q0_ref, q1_ref, g2_ref, d0_ref, d1_ref, b_ref, batch_ref,
              af_ref, fw1a_ref, fw1b_ref, fb1_ref, fw2_ref, fb2_ref,
              out_ref, sums, cnts):
  i = pl.program_id(0)
  dis = _dis(d0_ref[...], d1_ref[...], i)
  h2 = jnp.maximum(
      dis * (q0_ref[...] + q1_ref[...]) + dis * dis * g2_ref[...] + b_ref[...],
      0.0)
  row = lax.broadcasted_iota(jnp.int32, (_RB, 1), 0) + i * _RB
  h2 = jnp.where(row < _N, h2, 0.0)
  oh = (batch_ref[...] ==
        lax.broadcasted_iota(jnp.int32, (_B, _RB), 0)).astype(jnp.float32)
  psum = jnp.dot(oh, h2, preferred_element_type=jnp.float32)
  pcnt = jnp.sum(oh, axis=1, keepdims=True)

  @pl.when(i == 0)
  def _():
    sums[...] = jnp.zeros_like(sums)
    cnts[...] = jnp.zeros_like(cnts)

  sums[...] = sums[...] + psum
  cnts[...] = cnts[...] + pcnt

  @pl.when(i == _NPB - 1)
  def _():
    pooled = sums[...] / jnp.maximum(cnts[...], 1.0)
    z = jnp.maximum(
        jnp.dot(pooled, fw1a_ref[...], preferred_element_type=jnp.float32)
        + jnp.dot(af_ref[...], fw1b_ref[...], preferred_element_type=jnp.float32)
        + fb1_ref[...], 0.0)
    out_ref[...] = (jnp.dot(z, fw2_ref[...], preferred_element_type=jnp.float32)
                    + fb2_ref[...])


def _row_spec():
  return pl.BlockSpec((_RB, _H), lambda i: (i, 0))


def _col_spec():
  return pl.BlockSpec((_RB, 1), lambda i: (i, 0))


def _full_spec(shape):
  return pl.BlockSpec(shape, lambda i: tuple(0 for _ in shape))


def kernel(x, edge_index, batch, additional_features,
           W1, b1, W2, b2, FW1, Fb1, FW2, Fb2):
  f32 = jnp.float32
  xp = jnp.pad(x, ((0, _NP - _N), (0, 0)))
  pad_i = jnp.arange(_EP - _E, dtype=jnp.int32)
  src_pad = _N + pad_i % (_NP - _N)
  dst_pad = _N + (pad_i + 120) % (_NP - _N)
  src2d = jnp.concatenate([edge_index[0], src_pad]).reshape(_NW * _NBW, _EB)
  dst2d = jnp.concatenate([edge_index[1], dst_pad]).reshape(_NW * _NBW, _EB)
  batch2 = jnp.pad(batch, (0, _NP - _N), constant_values=_B)[None, :]
  ones_eb = jnp.ones((_EB,), f32)
  zeros_zr = jnp.zeros((_ZR,), f32)
  zrows = jnp.zeros((_ZR, _H), f32)
  b1r = b1[None, :]
  b2r = b2[None, :]
  fw1a = FW1[:_H]
  fw1b = FW1[_H:]
  fb1r = Fb1[None, :]
  fw2p = jnp.pad(FW2, ((0, 0), (0, _H - FW2.shape[1])))
  fb2p = jnp.pad(Fb2, (0, _H - Fb2.shape[0]))[None, :]

  p0b, p1b = _sc_deg(dst2d, ones_eb, zeros_zr)

  g1, ht1 = pl.pallas_call(
      _tc1_body,
      grid=(_NPB,),
      in_specs=[_row_spec(), _full_spec((_D, _H)), _row_spec(), _row_spec()],
      out_specs=[_row_spec(), _row_spec()],
      out_shape=[jax.ShapeDtypeStruct((_NP, _H), f32)] * 2,
  )(xp, W1, p0b, p1b)

  a0, a1 = _sc_agg(ht1, src2d, dst2d, zrows)

  g2, ht2 = pl.pallas_call(
      _tc2_body,
      grid=(_NPB,),
      in_specs=[_row_spec(), _row_spec(), _row_spec(), _row_spec(),
                _row_spec(), _full_spec((_H, _H)), _full_spec((1, _H))],
      out_specs=[_row_spec(), _row_spec()],
      out_shape=[jax.ShapeDtypeStruct((_NP, _H), f32)] * 2,
  )(a0, a1, g1, p0b, p1b, W2, b1r)

  q0, q1 = _sc_agg(ht2, src2d, dst2d, zrows)

  out_full = pl.pallas_call(
      _tc3_body,
      grid=(_NPB,),
      in_specs=[_row_spec(), _row_spec(), _row_spec(), _row_spec(),
                _row_spec(), _full_spec((1, _H)),
                pl.BlockSpec((1, _RB), lambda i: (0, i)),
                _full_spec((_B, _A)), _full_spec((_H, _H)),
                _full_spec((_A, _H)), _full_spec((1, _H)),
                _full_spec((_H, _H)), _full_spec((1, _H))],
      out_specs=pl.BlockSpec((_B, _H), lambda i: (0, 0)),
      out_shape=jax.ShapeDtypeStruct((_B, _H), f32),
      scratch_shapes=[pltpu.VMEM((_B, _H), f32), pltpu.VMEM((_B, 1), f32)],
  )(q0, q1, g2, p0b, p1b, b2r, batch2, additional_features,
    fw1a, fw1b, fb1r, fw2p, fb2p)

  return out_full[:, :FW2.shape[1]]


# speedup vs baseline: 1.0671x; 1.0048x over previous
import functools

import jax
import jax.numpy as jnp
from jax import lax
from jax.experimental import pallas as pl
from jax.experimental.pallas import tpu as pltpu
from jax.experimental.pallas import tpu_sc as plsc

_N = 10000
_E = 320000
_D = 128
_H = 128
_A = 16
_B = 64

_RB = 512
_NP = 10240
_NPB = _NP // _RB

_NC, _NS = 2, 16
_NW = _NC * _NS
_EB = 128
_NBW = 80
_NBW0 = 80
_NBW1 = 80
_IDXC = 40
_QB = _EB // 2
_EP = _NW * _NBW * _EB
_ZR = _NP // _NS

_mesh = plsc.VectorSubcoreMesh(
    core_axis_name="c", subcore_axis_name="s",
    num_cores=_NC, num_subcores=_NS)


def _sc_deg(dst2d, ones_eb, zeros_zr):
  @functools.partial(
      pl.kernel,
      out_type=[jax.ShapeDtypeStruct((_NP, _H), jnp.float32)] * 2,
      mesh=_mesh,
      scratch_types=[
          pltpu.VMEM((_NBW, _EB), jnp.int32),
          pltpu.VMEM((_EB,), jnp.float32),
          pltpu.VMEM((_ZR,), jnp.float32),
          pltpu.VMEM((_ZR, _H), jnp.float32),
          pltpu.VMEM_SHARED((_NP,), jnp.float32),
      ],
  )
  def run(dst_hbm, ones_hbm, z_hbm, out0_hbm, out1_hbm,
          dstv, onesv, dv, bc, acc):
    c = lax.axis_index("c")
    s = lax.axis_index("s")
    w = c * _NS + s
    pltpu.sync_copy(z_hbm, acc.at[pl.ds(s * _ZR, _ZR)])
    pltpu.sync_copy(dst_hbm.at[pl.ds(w * _NBW, _NBW)], dstv)
    pltpu.sync_copy(ones_hbm, onesv)
    plsc.subcore_barrier()

    def body(j, carry):
      pltpu.sync_copy(onesv, acc.at[dstv.at[j]], add=True)
      return carry

    lax.fori_loop(0, _NBW, body, 0)
    plsc.subcore_barrier()
    pltpu.sync_copy(acc.at[pl.ds(s * _ZR, _ZR)], dv)

    def brow(rg, carry):
      vals = dv[pl.ds(16 * rg, 16)]
      for k in range(16):
        row = jnp.full((16,), vals[k], jnp.float32)
        for q in range(8):
          bc[16 * rg + k, pl.ds(16 * q, 16)] = row
      return carry

    lax.fori_loop(0, _ZR // 16, brow, 0)

    @pl.when(c == 0)
    def _():
      pltpu.sync_copy(bc, out0_hbm.at[pl.ds(s * _ZR, _ZR)])

    @pl.when(c == 1)
    def _():
      pltpu.sync_copy(bc, out1_hbm.at[pl.ds(s * _ZR, _ZR)])

  return run(dst2d, ones_eb, zeros_zr)


def _sc_agg(ht, src2d, dst2d, zrows):
  @functools.partial(
      pl.kernel,
      out_type=[jax.ShapeDtypeStruct((_NP, _H), jnp.float32)] * 2,
      mesh=_mesh,
      scratch_types=[
          pltpu.VMEM((_IDXC, _EB), jnp.int32),
          pltpu.VMEM((_IDXC, _EB), jnp.int32),
          pltpu.VMEM((2 * _EB, _H), jnp.float32),
          pltpu.VMEM_SHARED((_NP, _H), jnp.float32),
          [pltpu.SemaphoreType.DMA] * 4,
      ],
  )
  def run(ht_hbm, src_hbm, dst_hbm, z_hbm, out0_hbm, out1_hbm,
          srcv, dstv, rb, acc, sems):
    c = lax.axis_index("c")
    s = lax.axis_index("s")
    base = jnp.where(c == 0, s * _NBW0, _NS * _NBW0 + s * _NBW1)
    nchunks = jnp.where(c == 0, _NBW0 // _IDXC, _NBW1 // _IDXC)
    pltpu.sync_copy(z_hbm, acc.at[pl.ds(s * _ZR, _ZR)])
    plsc.subcore_barrier()

    def issue(jj, par):
      s0 = 2 * par
      for q in range(2):
        pltpu.async_copy(ht_hbm.at[srcv.at[jj, pl.ds(q * _QB, _QB)]],
                         rb.at[pl.ds((s0 + q) * _QB, _QB)], sems[s0 + q])

    def drain_scatter(jj, par):
      s0 = 2 * par
      for q in range(2):
        pltpu.make_async_copy(ht_hbm.at[srcv.at[jj, pl.ds(q * _QB, _QB)]],
                              rb.at[pl.ds((s0 + q) * _QB, _QB)],
                              sems[s0 + q]).wait()
      pltpu.sync_copy(rb.at[pl.ds(s0 * _QB, _EB)], acc.at[dstv.at[jj]],
                      add=True)

    def chunk(p, carry):
      off = base + p * _IDXC
      pltpu.sync_copy(src_hbm.at[pl.ds(off, _IDXC)], srcv)
      pltpu.sync_copy(dst_hbm.at[pl.ds(off, _IDXC)], dstv)
      issue(0, 0)
      issue(1, 1)

      def group(g, carry2):
        b0 = 4 * g
        for k in range(4):
          jj = b0 + k
          drain_scatter(jj, k % 2)

          @pl.when(jj + 2 < _IDXC)
          def _():
            issue(jj + 2, k % 2)
        return carry2

      lax.fori_loop(0, _IDXC // 4, group, 0)
      return carry

    lax.fori_loop(0, nchunks, chunk, 0)
    plsc.subcore_barrier()

    @pl.when(c == 0)
    def _():
      pltpu.sync_copy(acc.at[pl.ds(s * _ZR, _ZR)],
                      out0_hbm.at[pl.ds(s * _ZR, _ZR)])

    @pl.when(c == 1)
    def _():
      pltpu.sync_copy(acc.at[pl.ds(s * _ZR, _ZR)],
                      out1_hbm.at[pl.ds(s * _ZR, _ZR)])

  return run(ht, src2d, dst2d, zrows)


def _dis(p0, p1, i):
  dis = lax.rsqrt(1.0 + p0 + p1)
  row = lax.broadcasted_iota(jnp.int32, (_RB, 1), 0) + i * _RB
  return jnp.where(row < _N, dis, 0.0)


def _tc1a_body(x_ref, w_ref, g_ref):
  g_ref[...] = jnp.dot(x_ref[...], w_ref[...],
                       preferred_element_type=jnp.float32)


def _tc1b_body(g_ref, d0_ref, d1_ref, ht_ref):
  i = pl.program_id(0)
  dis = _dis(d0_ref[...], d1_ref[...], i)
  ht_ref[...] = g_ref[...] * dis


def _tc2_body(a0_ref, a1_ref, g1_ref, d0_ref, d1_ref, w_ref, b_ref,
              g2_ref, ht_ref):
  i = pl.program_id(0)
  dis = _dis(d0_ref[...], d1_ref[...], i)
  h1 = jnp.maximum(
      dis * (a0_ref[...] + a1_ref[...]) + dis * dis * g1_ref[...] + b_ref[...],
      0.0)
  g2 = jnp.dot(h1, w_ref[...], preferred_element_type=jnp.float32)
  g2_ref[...] = g2
  ht_ref[...] = g2 * dis


def _tc3_body(q0_ref, q1_ref, g2_ref, d0_ref, d1_ref, b_ref, batch_ref,
              af_ref, fw1a_ref, fw1b_ref, fb1_ref, fw2_ref, fb2_ref,
              out_ref, sums, cnts):
  i = pl.program_id(0)
  dis = _dis(d0_ref[...], d1_ref[...], i)
  h2 = jnp.maximum(
      dis * (q0_ref[...] + q1_ref[...]) + dis * dis * g2_ref[...] + b_ref[...],
      0.0)
  row = lax.broadcasted_iota(jnp.int32, (_RB, 1), 0) + i * _RB
  h2 = jnp.where(row < _N, h2, 0.0)
  oh = (batch_ref[...] ==
        lax.broadcasted_iota(jnp.int32, (_B, _RB), 0)).astype(jnp.float32)
  psum = jnp.dot(oh, h2, preferred_element_type=jnp.float32)
  pcnt = jnp.sum(oh, axis=1, keepdims=True)

  @pl.when(i == 0)
  def _():
    sums[...] = jnp.zeros_like(sums)
    cnts[...] = jnp.zeros_like(cnts)

  sums[...] = sums[...] + psum
  cnts[...] = cnts[...] + pcnt

  @pl.when(i == _NPB - 1)
  def _():
    pooled = sums[...] / jnp.maximum(cnts[...], 1.0)
    z = jnp.maximum(
        jnp.dot(pooled, fw1a_ref[...], preferred_element_type=jnp.float32)
        + jnp.dot(af_ref[...], fw1b_ref[...], preferred_element_type=jnp.float32)
        + fb1_ref[...], 0.0)
    out_ref[...] = (jnp.dot(z, fw2_ref[...], preferred_element_type=jnp.float32)
                    + fb2_ref[...])


def _row_spec():
  return pl.BlockSpec((_RB, _H), lambda i: (i, 0))


def _col_spec():
  return pl.BlockSpec((_RB, 1), lambda i: (i, 0))


def _full_spec(shape):
  return pl.BlockSpec(shape, lambda i: tuple(0 for _ in shape))


def kernel(x, edge_index, batch, additional_features,
           W1, b1, W2, b2, FW1, Fb1, FW2, Fb2):
  f32 = jnp.float32
  xp = jnp.pad(x, ((0, _NP - _N), (0, 0)))
  pad_i = jnp.arange(_EP - _E, dtype=jnp.int32)
  src_pad = _N + pad_i % (_NP - _N)
  dst_pad = _N + (pad_i + 120) % (_NP - _N)
  src2d = jnp.concatenate([edge_index[0], src_pad]).reshape(_NW * _NBW, _EB)
  dst2d = jnp.concatenate([edge_index[1], dst_pad]).reshape(_NW * _NBW, _EB)
  batch2 = jnp.pad(batch, (0, _NP - _N), constant_values=_B)[None, :]
  ones_eb = jnp.ones((_EB,), f32)
  zeros_zr = jnp.zeros((_ZR,), f32)
  zrows = jnp.zeros((_ZR, _H), f32)
  b1r = b1[None, :]
  b2r = b2[None, :]
  fw1a = FW1[:_H]
  fw1b = FW1[_H:]
  fb1r = Fb1[None, :]
  fw2p = jnp.pad(FW2, ((0, 0), (0, _H - FW2.shape[1])))
  fb2p = jnp.pad(Fb2, (0, _H - Fb2.shape[0]))[None, :]

  p0b, p1b = _sc_deg(dst2d, ones_eb, zeros_zr)

  g1 = pl.pallas_call(
      _tc1a_body,
      grid=(_NPB,),
      in_specs=[_row_spec(), _full_spec((_D, _H))],
      out_specs=_row_spec(),
      out_shape=jax.ShapeDtypeStruct((_NP, _H), f32),
  )(xp, W1)

  ht1 = pl.pallas_call(
      _tc1b_body,
      grid=(_NPB,),
      in_specs=[_row_spec(), _row_spec(), _row_spec()],
      out_specs=_row_spec(),
      out_shape=jax.ShapeDtypeStruct((_NP, _H), f32),
  )(g1, p0b, p1b)

  a0, a1 = _sc_agg(ht1, src2d, dst2d, zrows)

  g2, ht2 = pl.pallas_call(
      _tc2_body,
      grid=(_NPB,),
      in_specs=[_row_spec(), _row_spec(), _row_spec(), _row_spec(),
                _row_spec(), _full_spec((_H, _H)), _full_spec((1, _H))],
      out_specs=[_row_spec(), _row_spec()],
      out_shape=[jax.ShapeDtypeStruct((_NP, _H), f32)] * 2,
  )(a0, a1, g1, p0b, p1b, W2, b1r)

  q0, q1 = _sc_agg(ht2, src2d, dst2d, zrows)

  out_full = pl.pallas_call(
      _tc3_body,
      grid=(_NPB,),
      in_specs=[_row_spec(), _row_spec(), _row_spec(), _row_spec(),
                _row_spec(), _full_spec((1, _H)),
                pl.BlockSpec((1, _RB), lambda i: (0, i)),
                _full_spec((_B, _A)), _full_spec((_H, _H)),
                _full_spec((_A, _H)), _full_spec((1, _H)),
                _full_spec((_H, _H)), _full_spec((1, _H))],
      out_specs=pl.BlockSpec((_B, _H), lambda i: (0, 0)),
      out_shape=jax.ShapeDtypeStruct((_B, _H), f32),
      scratch_shapes=[pltpu.VMEM((_B, _H), f32), pltpu.VMEM((_B, 1), f32)],
  )(q0, q1, g2, p0b, p1b, b2r, batch2, additional_features,
    fw1a, fw1b, fb1r, fw2p, fb2p)

  return out_full[:, :FW2.shape[1]]
